# baseline (speedup 1.0000x reference)
import functools

import jax
import jax.numpy as jnp
from jax import lax
from jax.experimental import pallas as pl
from jax.experimental.pallas import tpu as pltpu

_ROWS = 512
_COLS = 16384
_CHUNKS = [8, 8, 16, 32] + [64] * 6 + [32, 16, 8, 8]
_NBUF = 6
_BUFROWS = 64


def _row_starts():
    starts, r = [], 0
    for c in _CHUNKS:
        starts.append(r)
        r += c
    assert r == _ROWS
    return starts


_STARTS = _row_starts()


def _tc_body(x_hbm, t_hbm, o_ref, xbuf, tbuf, *sems):
    xsem = sems[:_NBUF]
    tsem = sems[_NBUF:]

    def issue(ci):
        s = ci % _NBUF
        r0, nr = _STARTS[ci], _CHUNKS[ci]
        pltpu.async_copy(x_hbm.at[pl.ds(r0, nr), :], xbuf.at[s, pl.ds(0, nr)], xsem[s])
        pltpu.async_copy(t_hbm.at[pl.ds(r0, nr), :], tbuf.at[s, pl.ds(0, nr)], tsem[s])

    for ci in range(_NBUF):
        issue(ci)

    total = jnp.zeros((1, 1), jnp.float32)
    for ci in range(len(_CHUNKS)):
        s = ci % _NBUF
        nr = _CHUNKS[ci]
        pltpu.make_async_copy(
            x_hbm.at[pl.ds(0, nr), :], xbuf.at[s, pl.ds(0, nr)], xsem[s]).wait()
        pltpu.make_async_copy(
            t_hbm.at[pl.ds(0, nr), :], tbuf.at[s, pl.ds(0, nr)], tsem[s]).wait()
        x = 0.5 * (1.0 + jnp.tanh(0.5 * xbuf[s, pl.ds(0, nr)]))
        t = tbuf[s, pl.ds(0, nr)]
        p = jnp.sum(x * t, axis=1)
        sm = jnp.sum(x, axis=1)
        ts = jnp.sum(t, axis=1)
        loss = 1.0 - (2.0 * p + 1.0) / (sm + ts + 1.0)
        total = total + jnp.sum(loss).reshape(1, 1)
        if ci + _NBUF < len(_CHUNKS):
            issue(ci + _NBUF)
    o_ref[...] = total


def kernel(inputs, targets, num_boxes):
    tc_out = pl.pallas_call(
        _tc_body,
        in_specs=[
            pl.BlockSpec(memory_space=pl.ANY),
            pl.BlockSpec(memory_space=pl.ANY),
        ],
        out_specs=pl.BlockSpec(memory_space=pltpu.VMEM),
        out_shape=jax.ShapeDtypeStruct((1, 1), jnp.float32),
        scratch_shapes=(
            [pltpu.VMEM((_NBUF, _BUFROWS, _COLS), jnp.float32)] * 2
            + [pltpu.SemaphoreType.DMA] * (2 * _NBUF)
        ),
    )(inputs, targets)
    return tc_out[0, 0] / (num_boxes + 1e-06)


# speedup vs baseline: 1.0433x; 1.0433x over previous
import functools

import jax
import jax.numpy as jnp
from jax import lax
from jax.experimental import pallas as pl
from jax.experimental.pallas import tpu as pltpu

_ROWS = 512
_COLS = 16384
_CHUNKS = [8, 8, 16, 32] + [64] * 6 + [32, 32]
_NBUF = 4
_BUFROWS = 64


def _row_starts():
    starts, r = [], 0
    for c in _CHUNKS:
        starts.append(r)
        r += c
    assert r == _ROWS
    return starts


_STARTS = _row_starts()


def _tc_body(x_hbm, t_hbm, o_ref, xbuf, tbuf, *sems):
    xsem = sems[:_NBUF]
    tsem = sems[_NBUF:]

    def issue(ci):
        s = ci % _NBUF
        r0, nr = _STARTS[ci], _CHUNKS[ci]
        pltpu.async_copy(x_hbm.at[pl.ds(r0, nr), :], xbuf.at[s, pl.ds(0, nr)], xsem[s])
        pltpu.async_copy(t_hbm.at[pl.ds(r0, nr), :], tbuf.at[s, pl.ds(0, nr)], tsem[s])

    for ci in range(_NBUF):
        issue(ci)

    total = jnp.zeros((1, 1), jnp.float32)
    for ci in range(len(_CHUNKS)):
        s = ci % _NBUF
        nr = _CHUNKS[ci]
        pltpu.make_async_copy(
            x_hbm.at[pl.ds(0, nr), :], xbuf.at[s, pl.ds(0, nr)], xsem[s]).wait()
        pltpu.make_async_copy(
            t_hbm.at[pl.ds(0, nr), :], tbuf.at[s, pl.ds(0, nr)], tsem[s]).wait()
        x = 0.5 * (1.0 + jnp.tanh(0.5 * xbuf[s, pl.ds(0, nr)]))
        t = tbuf[s, pl.ds(0, nr)]
        p = jnp.sum(x * t, axis=1)
        sm = jnp.sum(x, axis=1)
        ts = jnp.sum(t, axis=1)
        loss = 1.0 - (2.0 * p + 1.0) / (sm + ts + 1.0)
        total = total + jnp.sum(loss).reshape(1, 1)
        if ci + _NBUF < len(_CHUNKS):
            issue(ci + _NBUF)
    o_ref[...] = total


def kernel(inputs, targets, num_boxes):
    tc_out = pl.pallas_call(
        _tc_body,
        in_specs=[
            pl.BlockSpec(memory_space=pl.ANY),
            pl.BlockSpec(memory_space=pl.ANY),
        ],
        out_specs=pl.BlockSpec(memory_space=pltpu.VMEM),
        out_shape=jax.ShapeDtypeStruct((1, 1), jnp.float32),
        scratch_shapes=(
            [pltpu.VMEM((_NBUF, _BUFROWS, _COLS), jnp.float32)] * 2
            + [pltpu.SemaphoreType.DMA] * (2 * _NBUF)
        ),
    )(inputs, targets)
    return tc_out[0, 0] / (num_boxes + 1e-06)


# speedup vs baseline: 1.1239x; 1.0772x over previous
import functools

import jax
import jax.numpy as jnp
from jax import lax
from jax.experimental import pallas as pl
from jax.experimental.pallas import tpu as pltpu

_ROWS = 512
_COLS = 16384
_CHUNKS = [8, 8, 16, 32] + [64] * 6 + [32, 32]
_NBUF = 4
_BUFROWS = 64


def _row_starts():
    starts, r = [], 0
    for c in _CHUNKS:
        starts.append(r)
        r += c
    assert r == _ROWS
    return starts


_STARTS = _row_starts()


def _tc_body(nb_ref, x_hbm, t_hbm, o_ref, xbuf, tbuf, *sems):
    xsem = sems[:_NBUF]
    tsem = sems[_NBUF:]

    def issue(ci):
        s = ci % _NBUF
        r0, nr = _STARTS[ci], _CHUNKS[ci]
        pltpu.async_copy(x_hbm.at[pl.ds(r0, nr), :], xbuf.at[s, pl.ds(0, nr)], xsem[s])
        pltpu.async_copy(t_hbm.at[pl.ds(r0, nr), :], tbuf.at[s, pl.ds(0, nr)], tsem[s])

    for ci in range(_NBUF):
        issue(ci)

    total = jnp.zeros((1, 1), jnp.float32)
    for ci in range(len(_CHUNKS)):
        s = ci % _NBUF
        nr = _CHUNKS[ci]
        pltpu.make_async_copy(
            x_hbm.at[pl.ds(0, nr), :], xbuf.at[s, pl.ds(0, nr)], xsem[s]).wait()
        pltpu.make_async_copy(
            t_hbm.at[pl.ds(0, nr), :], tbuf.at[s, pl.ds(0, nr)], tsem[s]).wait()
        x = 0.5 * (1.0 + jnp.tanh(0.5 * xbuf[s, pl.ds(0, nr)]))
        t = tbuf[s, pl.ds(0, nr)]
        p = jnp.sum(x * t, axis=1)
        sm = jnp.sum(x, axis=1)
        ts = jnp.sum(t, axis=1)
        loss = 1.0 - (2.0 * p + 1.0) / (sm + ts + 1.0)
        total = total + jnp.sum(loss).reshape(1, 1)
        if ci + _NBUF < len(_CHUNKS):
            issue(ci + _NBUF)
    o_ref[...] = total / (nb_ref[0] + 1e-06)


def kernel(inputs, targets, num_boxes):
    nb = jnp.asarray(num_boxes, jnp.float32).reshape(1)
    tc_out = pl.pallas_call(
        _tc_body,
        in_specs=[
            pl.BlockSpec(memory_space=pltpu.SMEM),
            pl.BlockSpec(memory_space=pl.ANY),
            pl.BlockSpec(memory_space=pl.ANY),
        ],
        out_specs=pl.BlockSpec(memory_space=pltpu.VMEM),
        out_shape=jax.ShapeDtypeStruct((1, 1), jnp.float32),
        scratch_shapes=(
            [pltpu.VMEM((_NBUF, _BUFROWS, _COLS), jnp.float32)] * 2
            + [pltpu.SemaphoreType.DMA] * (2 * _NBUF)
        ),
    )(nb, inputs, targets)
    return tc_out[0, 0]
